# baseline (speedup 1.0000x reference)
import functools

import jax
import jax.numpy as jnp
from jax import lax
from jax.experimental import pallas as pl
from jax.experimental.pallas import tpu as pltpu
from jax.experimental.pallas import tpu_sc as plsc

_SC_CORES = 2
_SC_SUBCORES = 16
_SC_WORKERS = _SC_CORES * _SC_SUBCORES
_GATHER_CHUNK = 128
_ROW = 128

_QB = 256


def _top2_body(q_ref, k_ref, i1_ref, i2_ref, g_ref, tab_ref):
    b = pl.program_id(0)
    qb = pl.program_id(1)
    n = k_ref.shape[2]
    x_q = q_ref[0, :, 0:1]
    y_q = q_ref[0, :, 1:2]
    z_q = q_ref[0, :, 2:3]
    x_k = k_ref[0, 0:1, :]
    y_k = k_ref[0, 1:2, :]
    z_k = k_ref[0, 2:3, :]
    tab_ref[0] = jnp.zeros((_QB, _ROW), jnp.float32)
    tab_ref[0, :, 0:3] = q_ref[0]
    dx = x_q - x_k
    d2 = dx * dx
    dy = y_q - y_k
    d2 = d2 + dy * dy
    dz = z_q - z_k
    d2 = d2 + dz * dz
    d = jnp.sqrt(d2)
    col = lax.broadcasted_iota(jnp.int32, (_QB, n), 1)
    row = lax.broadcasted_iota(jnp.int32, (_QB, n), 0) + qb * _QB
    d = jnp.where(col == row, jnp.float32(1e9), d)
    m1 = jnp.min(d, axis=1, keepdims=True)
    i1 = jnp.min(jnp.where(d == m1, col, n), axis=1, keepdims=True)
    d_rest = jnp.where(col == i1, jnp.float32(jnp.inf), d)
    m2 = jnp.min(d_rest, axis=1, keepdims=True)
    i2 = jnp.min(jnp.where(d_rest == m2, col, n), axis=1, keepdims=True)
    i1_ref[0] = i1
    i2_ref[0] = i2
    base = b * n
    g_ref[0, :, 0:1] = i1 + base
    g_ref[0, :, 1:2] = i2 + base


def _tc_top2(coords, coords_k):
    bsz, n, _ = coords.shape
    grid = (bsz, n // _QB)
    return pl.pallas_call(
        _top2_body,
        grid=grid,
        in_specs=[
            pl.BlockSpec((1, _QB, 3), lambda b, q: (b, q, 0)),
            pl.BlockSpec((1, 8, n), lambda b, q: (b, 0, 0)),
        ],
        out_specs=[
            pl.BlockSpec((1, _QB, 1), lambda b, q: (b, q, 0)),
            pl.BlockSpec((1, _QB, 1), lambda b, q: (b, q, 0)),
            pl.BlockSpec((1, _QB, 2), lambda b, q: (b, q, 0)),
            pl.BlockSpec((1, _QB, _ROW), lambda b, q: (b, q, 0)),
        ],
        out_shape=[
            jax.ShapeDtypeStruct((bsz, n, 1), jnp.int32),
            jax.ShapeDtypeStruct((bsz, n, 1), jnp.int32),
            jax.ShapeDtypeStruct((bsz, n, 2), jnp.int32),
            jax.ShapeDtypeStruct((bsz, n, _ROW), jnp.float32),
        ],
    )(coords, coords_k)


def _sc_gather(table, idx3d):
    chunks = idx3d.shape[1]
    per_worker = chunks * _GATHER_CHUNK
    total = _SC_WORKERS * per_worker
    mesh = plsc.VectorSubcoreMesh(core_axis_name="c", subcore_axis_name="s")

    @functools.partial(
        pl.kernel,
        mesh=mesh,
        out_type=jax.ShapeDtypeStruct((total, _ROW), jnp.float32),
        scratch_types=[
            pltpu.VMEM((chunks, _GATHER_CHUNK), jnp.int32),
            pltpu.VMEM((per_worker, _ROW), jnp.float32),
            pltpu.SemaphoreType.DMA,
        ],
    )
    def gather_kernel(table_hbm, idx_hbm, out_hbm, idx_v, rows_v, sem):
        wid = lax.axis_index("s") * _SC_CORES + lax.axis_index("c")
        pltpu.sync_copy(idx_hbm.at[wid], idx_v)
        copies = []
        for c in range(chunks):
            copies.append(
                pltpu.async_copy(
                    table_hbm.at[idx_v.at[c]],
                    rows_v.at[pl.ds(c * _GATHER_CHUNK, _GATHER_CHUNK)],
                    sem,
                )
            )
        for cp in copies:
            cp.wait()
        pltpu.sync_copy(rows_v, out_hbm.at[pl.ds(wid * per_worker, per_worker)])

    return gather_kernel(table, idx3d)


def kernel(coords, mask):
    del mask
    bsz, n, _ = coords.shape
    coords_k = jnp.transpose(
        jnp.pad(coords, ((0, 0), (0, 0), (0, 5))), (0, 2, 1)
    )
    halves = []
    hb = bsz // 2
    for h in range(2):
        sl = slice(h * hb, (h + 1) * hb)
        i1, i2, g, table = _tc_top2(coords[sl], coords_k[sl])
        chunks = (hb * n * 2) // (_SC_WORKERS * _GATHER_CHUNK)
        idx3d = g.reshape(_SC_WORKERS, chunks, _GATHER_CHUNK)
        rows = _sc_gather(table.reshape(hb * n, _ROW), idx3d)
        halves.append((i1, i2, rows.reshape(hb, n, 2, _ROW)))
    i1 = jnp.concatenate([h[0] for h in halves], axis=0).reshape(bsz, n)
    i2 = jnp.concatenate([h[1] for h in halves], axis=0).reshape(bsz, n)
    rows = jnp.concatenate([h[2] for h in halves], axis=0)
    a = rows[:, :, 0, :3]
    c = rows[:, :, 1, :3]
    return a, c, i1, i2


# speedup vs baseline: 1.2049x; 1.2049x over previous
import dataclasses
import functools

import jax
import jax.numpy as jnp
from jax import lax
from jax.experimental import pallas as pl
from jax.experimental.pallas import tpu as pltpu
from jax.experimental.pallas import tpu_sc as plsc

_SC_CORES = 2
_SC_SUBCORES = 16
_SC_WORKERS = _SC_CORES * _SC_SUBCORES
_SC_LANES = 16

_QB = 256
_KC = 128


def _top2_body(q_ref, k_ref, i1_ref, i2_ref, ga_ref, gc_ref):
    b = pl.program_id(0)
    qb = pl.program_id(1)
    n = k_ref.shape[2]
    x_q = q_ref[0, :, 0:1]
    y_q = q_ref[0, :, 1:2]
    z_q = q_ref[0, :, 2:3]

    lane = lax.broadcasted_iota(jnp.int32, (_QB, _KC), 1)
    row = lax.broadcasted_iota(jnp.int32, (_QB, _KC), 0) + qb * _QB
    inf = jnp.float32(jnp.inf)
    m1 = jnp.full((_QB, _KC), inf)
    m2 = jnp.full((_QB, _KC), inf)
    i1 = jnp.full((_QB, _KC), n, jnp.int32)
    i2 = jnp.full((_QB, _KC), n, jnp.int32)
    for c in range(n // _KC):
        x_k = k_ref[0, 0:1, pl.ds(c * _KC, _KC)]
        y_k = k_ref[0, 1:2, pl.ds(c * _KC, _KC)]
        z_k = k_ref[0, 2:3, pl.ds(c * _KC, _KC)]
        dx = x_q - x_k
        d2 = dx * dx
        dy = y_q - y_k
        d2 = d2 + dy * dy
        dz = z_q - z_k
        d2 = d2 + dz * dz
        v = jnp.sqrt(d2)
        ci = lane + c * _KC
        v = jnp.where(ci == row, jnp.float32(1e9), v)
        take1 = v < m1
        take2 = v < m2
        m2n = jnp.where(take2, v, m2)
        i2n = jnp.where(take2, ci, i2)
        m2 = jnp.where(take1, m1, m2n)
        i2 = jnp.where(take1, i1, i2n)
        m1 = jnp.where(take1, v, m1)
        i1 = jnp.where(take1, ci, i1)

    big1 = jnp.min(m1, axis=1, keepdims=True)
    big_i1 = jnp.min(jnp.where(m1 == big1, i1, n), axis=1, keepdims=True)
    win = i1 == big_i1
    vals2 = jnp.where(win, m2, m1)
    idx2 = jnp.where(win, i2, i1)
    big2 = jnp.min(vals2, axis=1, keepdims=True)
    big_i2 = jnp.min(jnp.where(vals2 == big2, idx2, n), axis=1, keepdims=True)

    i1_ref[0] = big_i1
    i2_ref[0] = big_i2
    base = b * n
    ga_ref[0] = big_i1 + base
    gc_ref[0] = big_i2 + base


def _tc_top2(coords, coords_k):
    bsz, n, _ = coords.shape
    grid = (bsz, n // _QB)
    ispec = pl.BlockSpec((1, _QB, 1), lambda b, q: (b, q, 0))
    ishape = jax.ShapeDtypeStruct((bsz, n, 1), jnp.int32)
    return pl.pallas_call(
        _top2_body,
        grid=grid,
        in_specs=[
            pl.BlockSpec((1, _QB, 3), lambda b, q: (b, q, 0)),
            pl.BlockSpec((1, 8, n), lambda b, q: (b, 0, 0)),
        ],
        out_specs=[ispec, ispec, ispec, ispec],
        out_shape=[ishape, ishape, ishape, ishape],
    )(coords, coords_k)


def _sc_gather(table_t, idx_a, idx_c):
    rows = idx_a.shape[1]
    per_worker = rows * 128
    v = table_t.shape[1]
    mesh = plsc.VectorSubcoreMesh(core_axis_name="c", subcore_axis_name="s")
    out_t = jax.ShapeDtypeStruct((_SC_WORKERS, 3, per_worker), jnp.float32)

    cp = pltpu.CompilerParams()
    if "needs_layout_passes" in pltpu.CompilerParams.__dataclass_fields__:
        cp = dataclasses.replace(cp, needs_layout_passes=False)

    @functools.partial(
        pl.kernel,
        mesh=mesh,
        out_type=[out_t, out_t],
        scratch_types=[
            pltpu.VMEM((3, v), jnp.float32),
            pltpu.VMEM((rows, 128), jnp.int32),
            pltpu.VMEM((rows, 128), jnp.int32),
            pltpu.VMEM((3, per_worker), jnp.float32),
            pltpu.VMEM((3, per_worker), jnp.float32),
            pltpu.SemaphoreType.DMA,
        ],
        compiler_params=cp,
    )
    def gather_kernel(tab_hbm, ia_hbm, ic_hbm, oa_hbm, oc_hbm,
                      tab_v, ia_v, ic_v, ba_v, bc_v, sem):
        wid = lax.axis_index("s") * _SC_CORES + lax.axis_index("c")
        tab_cp = pltpu.async_copy(tab_hbm, tab_v, sem)
        pltpu.sync_copy(ia_hbm.at[wid], ia_v)
        pltpu.sync_copy(ic_hbm.at[wid], ic_v)
        tab_cp.wait()
        for idx_v, buf_v in ((ia_v, ba_v), (ic_v, bc_v)):
            for r in range(rows):
                for g in range(128 // _SC_LANES):
                    iv = idx_v[r, pl.ds(g * _SC_LANES, _SC_LANES)]
                    pos = r * 128 + g * _SC_LANES
                    for c in range(3):
                        cv = jnp.full((_SC_LANES,), c, jnp.int32)
                        vals = plsc.load_gather(tab_v, [cv, iv])
                        buf_v[c, pl.ds(pos, _SC_LANES)] = vals
        pltpu.sync_copy(ba_v, oa_hbm.at[wid])
        pltpu.sync_copy(bc_v, oc_hbm.at[wid])

    return gather_kernel(table_t, idx_a, idx_c)


def kernel(coords, mask):
    del mask
    bsz, n, _ = coords.shape
    coords_k = jnp.transpose(
        jnp.pad(coords, ((0, 0), (0, 0), (0, 5))), (0, 2, 1)
    )
    i1, i2, ga, gc = _tc_top2(coords, coords_k)
    table_t = jnp.transpose(coords.reshape(bsz * n, 3))
    rows = (bsz * n) // (_SC_WORKERS * 128)
    out_a, out_c = _sc_gather(
        table_t,
        ga.reshape(_SC_WORKERS, rows, 128),
        gc.reshape(_SC_WORKERS, rows, 128),
    )
    a = jnp.transpose(out_a, (0, 2, 1)).reshape(bsz, n, 3)
    c = jnp.transpose(out_c, (0, 2, 1)).reshape(bsz, n, 3)
    return a, c, i1.reshape(bsz, n), i2.reshape(bsz, n)
